# baseline (speedup 1.0000x reference)
import functools

import jax
import jax.numpy as jnp
from jax import lax
from jax.experimental import pallas as pl
from jax.experimental.pallas import tpu as pltpu, tpu_sc as plsc

NUM_CORES = 2
NUM_SUBCORES = 16
LANES = 16
NUM_WORKERS = NUM_CORES * NUM_SUBCORES

BATCH = 4
SEQ_LEN = 2048
D_MODEL = 1024
N_TOK = BATCH * SEQ_LEN
TOK_PER_W = N_TOK // NUM_WORKERS
CHUNK = 16
N_CHUNKS = TOK_PER_W // CHUNK
NB = 2
NGRP = N_CHUNKS // NB
SCALE = 32.0


@functools.partial(
    pl.kernel,
    out_type=jax.ShapeDtypeStruct((N_TOK, D_MODEL), jnp.float32),
    mesh=plsc.VectorSubcoreMesh(core_axis_name="c", subcore_axis_name="s"),
    scratch_types=[
        pltpu.VMEM((TOK_PER_W,), jnp.int32),
        pltpu.VMEM((NB, CHUNK, D_MODEL), jnp.float32),
        pltpu.VMEM((NB, CHUNK, D_MODEL), jnp.float32),
        pltpu.VMEM((NB, CHUNK, D_MODEL), jnp.float32),
        pltpu.SemaphoreType.DMA((NB,)),
        pltpu.SemaphoreType.DMA((NB,)),
        pltpu.SemaphoreType.DMA((NB,)),
    ],
)
def _emb_sc(x_hbm, emb1_hbm, emb2_hbm, out_hbm,
            idx_v, g_v, p_v, o_v, sem_g, sem_p, sem_o):
    wid = lax.axis_index("s") * NUM_CORES + lax.axis_index("c")
    base = wid * TOK_PER_W
    pos_base = lax.rem(base, SEQ_LEN)

    pltpu.sync_copy(x_hbm.at[pl.ds(base, TOK_PER_W)], idx_v)

    def start_in(c, b):
        pltpu.async_copy(
            emb1_hbm.at[idx_v.at[pl.ds(c * CHUNK, CHUNK)]],
            g_v.at[b], sem_g.at[b])
        pltpu.async_copy(
            emb2_hbm.at[pl.ds(pos_base + c * CHUNK, CHUNK)],
            p_v.at[b], sem_p.at[b])

    def wait_in(c, b):
        pltpu.make_async_copy(
            emb1_hbm.at[idx_v.at[pl.ds(c * CHUNK, CHUNK)]],
            g_v.at[b], sem_g.at[b]).wait()
        pltpu.make_async_copy(
            emb2_hbm.at[pl.ds(pos_base + c * CHUNK, CHUNK)],
            p_v.at[b], sem_p.at[b]).wait()

    def start_out(c, b):
        pltpu.async_copy(
            o_v.at[b], out_hbm.at[pl.ds(base + c * CHUNK, CHUNK)], sem_o.at[b])

    def wait_out(b):
        pltpu.make_async_copy(
            o_v.at[b], out_hbm.at[pl.ds(base, CHUNK)], sem_o.at[b]).wait()

    for b in range(NB):
        start_in(b, b)

    @pl.loop(0, NGRP)
    def _grp(grp):
        for b in range(NB):
            c = grp * NB + b
            wait_in(c, b)

            @pl.when(grp >= 1)
            def _():
                wait_out(b)

            @plsc.parallel_loop(0, CHUNK)
            def row_body(i):
                g_row = g_v.at[b].at[i]
                p_row = p_v.at[b].at[i]
                o_row = o_v.at[b].at[i]
                for k in range(D_MODEL // LANES):
                    sl = pl.ds(k * LANES, LANES)
                    o_row[sl] = g_row[sl] * SCALE + p_row[sl]

            @pl.when(grp < NGRP - 1)
            def _():
                start_in(c + NB, b)

            start_out(c, b)

    for b in range(NB):
        wait_out(b)


def kernel(x, emb1, emb2):
    xf = x.reshape(-1).astype(jnp.int32)
    out = _emb_sc(xf, emb1, emb2)
    return out.reshape(x.shape[0], x.shape[1], emb1.shape[1])
